# baseline (speedup 1.0000x reference)
import functools

import jax
import jax.numpy as jnp
from jax import lax
from jax.experimental import pallas as pl
from jax.experimental.pallas import tpu as pltpu
from jax.experimental.pallas import tpu_sc as plsc


def _log2(n):
    return n.bit_length() - 1


def _make_sc_kernel(B, D, C, end_size):
    mesh = plsc.VectorSubcoreMesh(
        core_axis_name="c", subcore_axis_name="s", num_cores=2, num_subcores=16)
    n_workers = 32
    rpw = (B * D) // n_workers
    nrows = rpw * D
    group_rows = B * D * D
    L = 16

    @functools.partial(
        pl.kernel,
        out_type=jax.ShapeDtypeStruct((3 * group_rows, C), jnp.float32),
        mesh=mesh,
        compiler_params=pltpu.CompilerParams(use_tc_tiling_on_sc=False),
        scratch_types=dict(
            snv=pltpu.VMEM((L,), jnp.int32),
            idx=pltpu.VMEM((4, 128), jnp.int32),
            bA0=pltpu.VMEM((nrows, C), jnp.float32),
            bA1=pltpu.VMEM((nrows, C), jnp.float32),
            bB0=pltpu.VMEM((nrows, C), jnp.float32),
            bC0=pltpu.VMEM((nrows, C), jnp.float32),
            bC1=pltpu.VMEM((nrows, C), jnp.float32),
            oA=pltpu.VMEM((nrows, C), jnp.float32),
            oB=pltpu.VMEM((nrows, C), jnp.float32),
            oC=pltpu.VMEM((nrows, C), jnp.float32),
            sem0=pltpu.SemaphoreType.DMA,
            sem1=pltpu.SemaphoreType.DMA,
            sem2=pltpu.SemaphoreType.DMA,
            sem3=pltpu.SemaphoreType.DMA,
        ),
    )
    def sc_kernel(vol, snum, out, snv, idx, bA0, bA1, bB0, bC0, bC1,
                  oA, oB, oC, sem0, sem1, sem2, sem3):
        cid = lax.axis_index("c")
        sid = lax.axis_index("s")
        wid = sid * 2 + cid
        wpb = D // rpw
        b = lax.shift_right_logical(wid, _log2(wpb))
        i0 = (wid & (wpb - 1)) * rpw

        pltpu.sync_copy(snum, snv)
        snvec = snv[:]
        sn = snvec[0]
        s = lax.shift_right_logical(sn * D, _log2(end_size))

        bbase = b * (D * D * D)

        rA0 = bbase + s * (D * D) + i0 * D
        cA0 = pltpu.async_copy(vol.at[pl.ds(rA0, nrows)], bA0, sem0)
        cA1 = pltpu.async_copy(vol.at[pl.ds(rA0 + D * D, nrows)], bA1, sem0)

        cBs = []
        for a in range(rpw):
            rB = bbase + (i0 + a) * (D * D) + s * D
            cBs.append(pltpu.async_copy(
                vol.at[pl.ds(rB, D)], bB0.at[pl.ds(a * D, D), :], sem1))

        iota = lax.iota(jnp.int32, L)
        for t in range(2):
            for c in range(8):
                e0 = t * 128 + c * 16
                a_c = e0 // D
                j0 = e0 - a_c * D
                base = bbase + (i0 + a_c) * (D * D) + j0 * D + s
                vec = base + iota * D
                idx[t, pl.ds(c * 16, 16)] = vec
                idx[t + 2, pl.ds(c * 16, 16)] = vec + 1
        cC0a = pltpu.async_copy(vol.at[idx.at[0]], bC0.at[pl.ds(0, 128), :], sem2)
        cC0b = pltpu.async_copy(vol.at[idx.at[1]], bC0.at[pl.ds(128, 128), :], sem2)
        cC1a = pltpu.async_copy(vol.at[idx.at[2]], bC1.at[pl.ds(0, 128), :], sem2)
        cC1b = pltpu.async_copy(vol.at[idx.at[3]], bC1.at[pl.ds(128, 128), :], sem2)

        fvec = ((snvec * D) & (end_size - 1)).astype(jnp.float32) * (1.0 / end_size)
        gvec = fvec + 1.0

        cA0.wait()
        cA1.wait()
        for c in cBs:
            c.wait()
        cC0a.wait()
        cC0b.wait()
        cC1a.wait()
        cC1b.wait()

        def body(k, carry):
            r0 = bA0[k, :]
            r1 = bA1[k, :]
            rb = bB0[k, :]
            c0 = bC0[k, :]
            c1 = bC1[k, :]
            oA[k, :] = gvec * (r1 - r0)
            oB[k, :] = gvec * (r1 - rb)
            oC[k, :] = fvec * (c1 - c0) + c0
            return carry
        lax.fori_loop(0, nrows, body, 0)

        obase = b * (D * D) + i0 * D
        wA = pltpu.async_copy(oA, out.at[pl.ds(obase, nrows)], sem3)
        wB = pltpu.async_copy(oB, out.at[pl.ds(group_rows + obase, nrows)], sem3)
        wC = pltpu.async_copy(oC, out.at[pl.ds(2 * group_rows + obase, nrows)], sem3)
        wA.wait()
        wB.wait()
        wC.wait()

    return sc_kernel


def kernel(volume, slice_ax, slice_num, upsmp):
    B, D, _, _, C = volume.shape
    end_size = D * 4
    sc = _make_sc_kernel(B, D, C, end_size)
    vol2 = volume.reshape(B * D * D * D, C)
    sn16 = jnp.broadcast_to(slice_num.reshape(-1)[:1].astype(jnp.int32), (16,))
    out = sc(vol2, sn16)
    return out.reshape(3 * B, D, D, C)
